# baseline (speedup 1.0000x reference)
import functools

import jax
import jax.numpy as jnp
from jax import lax
from jax.experimental import pallas as pl
from jax.experimental.pallas import tpu as pltpu
from jax.experimental.pallas import tpu_sc as plsc

B, S, D = 1, 2048, 2048
E, TOPK, FF = 8, 2, 1024
T = B * S
BR = 256
NB = (T * TOPK) // BR + (E - 1)
NPAD = NB * BR
NBP = 128
CK = 256

NC, NS = 2, 16
NW = NC * NS
TPW = T // NW
CH = 8
NCK = TPW // CH


def _router_body(x_ref, gw_ref, logits_ref, w0_ref, w1_ref,
                 s0_ref, s1_ref, meta_ref):
    x = x_ref[...]
    gw = gw_ref[...]
    logits = lax.dot_general(x, gw, (((1,), (1,)), ((), ())),
                             preferred_element_type=jnp.float32)
    logits_ref[...] = logits
    m = jnp.max(logits, axis=1, keepdims=True)
    ex = jnp.exp(logits - m)
    p = ex / jnp.sum(ex, axis=1, keepdims=True)
    a1 = jnp.argmax(p, axis=1).astype(jnp.int32)
    w1 = jnp.max(p, axis=1)
    cols = lax.broadcasted_iota(jnp.int32, p.shape, 1)
    p2 = jnp.where(cols == a1[:, None], -jnp.inf, p)
    a2 = jnp.argmax(p2, axis=1).astype(jnp.int32)
    w2 = jnp.max(p2, axis=1)
    w0_ref[...] = w1.reshape(T // 16, 16)
    w1_ref[...] = w2.reshape(T // 16, 16)

    oh1 = (cols == a1[:, None]).astype(jnp.float32)
    oh2 = (cols == a2[:, None]).astype(jnp.float32)
    cmb = oh1 + oh2

    r_i = lax.broadcasted_iota(jnp.int32, (CK, CK), 0)
    c_i = lax.broadcasted_iota(jnp.int32, (CK, CK), 1)
    tri = (r_i >= c_i).astype(jnp.float32)
    parts = []
    run = jnp.zeros((1, E), jnp.float32)
    for i in range(T // CK):
        blk = cmb[i * CK:(i + 1) * CK, :]
        s = lax.dot_general(tri, blk, (((1,), (0,)), ((), ())),
                            preferred_element_type=jnp.float32) + run
        parts.append(s)
        run = s[CK - 1:CK, :]
    csum = jnp.concatenate(parts, axis=0)
    excl = csum - cmb

    counts = run
    padded = jnp.ceil(counts / BR) * BR
    e_r = lax.broadcasted_iota(jnp.int32, (E, E), 0)
    e_c = lax.broadcasted_iota(jnp.int32, (E, E), 1)
    m8 = (e_r <= e_c).astype(jnp.float32)
    csum_p = lax.dot_general(padded, m8, (((1,), (0,)), ((), ())),
                             preferred_element_type=jnp.float32)
    offs_p = csum_p - padded

    base = excl + offs_p
    s0 = jnp.sum(oh1 * base, axis=1)
    s1 = jnp.sum(oh2 * base, axis=1)
    s0_ref[...] = s0.reshape(T // CH, CH).astype(jnp.int32)
    s1_ref[...] = s1.reshape(T // CH, CH).astype(jnp.int32)

    total = csum_p[0:1, E - 1:E]
    bstart = (lax.broadcasted_iota(jnp.int32, (NBP, 1), 0)
              .astype(jnp.float32) * BR)
    be = jnp.sum((bstart >= csum_p).astype(jnp.float32), axis=1,
                 keepdims=True)
    act = (bstart < total).astype(jnp.float32)
    be_last = jnp.sum((csum_p <= total - 1.0).astype(jnp.float32),
                      axis=1, keepdims=True)
    be_last = jnp.minimum(be_last, float(E - 1))
    be = jnp.where(act > 0, be, be_last)
    meta_ref[...] = jnp.concatenate([be, act], axis=1).astype(jnp.int32)


def _router(x, gate_w):
    return pl.pallas_call(
        _router_body,
        out_shape=(
            jax.ShapeDtypeStruct((T, E), jnp.float32),
            jax.ShapeDtypeStruct((T // 16, 16), jnp.float32),
            jax.ShapeDtypeStruct((T // 16, 16), jnp.float32),
            jax.ShapeDtypeStruct((T // CH, CH), jnp.int32),
            jax.ShapeDtypeStruct((T // CH, CH), jnp.int32),
            jax.ShapeDtypeStruct((NBP, 2), jnp.int32),
        ),
    )(x, gate_w)


@functools.lru_cache(maxsize=None)
def _sc_dispatch_kernel():
    mesh = plsc.VectorSubcoreMesh(core_axis_name="c", subcore_axis_name="s")

    @functools.partial(
        pl.kernel, mesh=mesh,
        out_type=jax.ShapeDtypeStruct((NPAD, D), jnp.float32),
        scratch_types=[
            pltpu.VMEM((NCK, CH), jnp.int32),
            pltpu.VMEM((NCK, CH), jnp.int32),
            pltpu.VMEM((CH, D), jnp.float32),
            pltpu.VMEM((CH, D), jnp.float32),
            pltpu.VMEM((CH, D), jnp.float32),
            pltpu.SemaphoreType.DMA,
            pltpu.SemaphoreType.DMA,
            pltpu.SemaphoreType.DMA,
            pltpu.SemaphoreType.DMA,
            pltpu.SemaphoreType.DMA,
            pltpu.SemaphoreType.DMA,
            pltpu.SemaphoreType.DMA,
            pltpu.SemaphoreType.DMA,
            pltpu.SemaphoreType.DMA,
        ],
    )
    def body(x_hbm, s0_hbm, s1_hbm, out_hbm, i0_v, i1_v,
             r0, r1, r2, g0, g1, g2, sa0, sa1, sa2, sb0, sb1, sb2):
        wid = lax.axis_index("s") * NC + lax.axis_index("c")
        base = wid * TPW
        rbase = wid * NCK
        pltpu.sync_copy(s0_hbm.at[pl.ds(rbase, NCK), :], i0_v)
        pltpu.sync_copy(s1_hbm.at[pl.ds(rbase, NCK), :], i1_v)
        bufs = (r0, r1, r2)
        gsems = (g0, g1, g2)
        sasems = (sa0, sa1, sa2)
        sbsems = (sb0, sb1, sb2)

        def rd(c):
            return pltpu.async_copy(
                x_hbm.at[pl.ds(base + c * CH, CH), :],
                bufs[c % 3], gsems[c % 3])

        def wr(c):
            return (pltpu.async_copy(bufs[c % 3], out_hbm.at[i0_v.at[c]],
                                     sasems[c % 3]),
                    pltpu.async_copy(bufs[c % 3], out_hbm.at[i1_v.at[c]],
                                     sbsems[c % 3]))

        gd = {0: rd(0)}
        sd = {}
        for c in range(NCK):
            if c + 1 < NCK:
                if c - 2 in sd:
                    sd[c - 2][0].wait()
                    sd[c - 2][1].wait()
                gd[c + 1] = rd(c + 1)
            gd[c].wait()
            sd[c] = wr(c)
        for c in (NCK - 3, NCK - 2, NCK - 1):
            sd[c][0].wait()
            sd[c][1].wait()

    return body


def _sc_dispatch(x, s0r, s1r):
    return _sc_dispatch_kernel()(x, s0r, s1r)


@functools.lru_cache(maxsize=None)
def _sc_combine_kernel():
    mesh = plsc.VectorSubcoreMesh(core_axis_name="c", subcore_axis_name="s")

    @functools.partial(
        pl.kernel, mesh=mesh,
        out_type=jax.ShapeDtypeStruct((T, D), jnp.float32),
        scratch_types=[
            pltpu.VMEM((NCK, CH), jnp.int32),
            pltpu.VMEM((NCK, CH), jnp.int32),
            pltpu.VMEM((TPW // 16, 16), jnp.float32),
            pltpu.VMEM((TPW // 16, 16), jnp.float32),
            pltpu.VMEM((CH, D), jnp.float32),
            pltpu.VMEM((CH, D), jnp.float32),
            pltpu.VMEM((CH, D), jnp.float32),
            pltpu.VMEM((CH, D), jnp.float32),
            pltpu.VMEM((CH, D), jnp.float32),
            pltpu.SemaphoreType.DMA,
            pltpu.SemaphoreType.DMA,
            pltpu.SemaphoreType.DMA,
            pltpu.SemaphoreType.DMA,
            pltpu.SemaphoreType.DMA,
            pltpu.SemaphoreType.DMA,
            pltpu.SemaphoreType.DMA,
            pltpu.SemaphoreType.DMA,
        ],
    )
    def body(y_hbm, s0_hbm, s1_hbm, w0_hbm, w1_hbm, out_hbm, i0_v, i1_v,
             w0_v, w1_v, a0, a1, a2, b0, b1,
             ga0, ga1, ga2, gb0, gb1, sa0, sa1, sa2):
        wid = lax.axis_index("s") * NC + lax.axis_index("c")
        base = wid * TPW
        rbase = wid * NCK
        qbase = wid * (TPW // 16)
        pltpu.sync_copy(s0_hbm.at[pl.ds(rbase, NCK), :], i0_v)
        pltpu.sync_copy(s1_hbm.at[pl.ds(rbase, NCK), :], i1_v)
        pltpu.sync_copy(w0_hbm.at[pl.ds(qbase, TPW // 16), :], w0_v)
        pltpu.sync_copy(w1_hbm.at[pl.ds(qbase, TPW // 16), :], w1_v)
        abufs = (a0, a1, a2)
        bbufs = (b0, b1)
        gasems = (ga0, ga1, ga2)
        gbsems = (gb0, gb1)
        sasems = (sa0, sa1, sa2)

        def gath_a(c):
            return pltpu.async_copy(
                y_hbm.at[i0_v.at[c]],
                abufs[c % 3], gasems[c % 3])

        def gath_b(c):
            return pltpu.async_copy(
                y_hbm.at[i1_v.at[c]],
                bbufs[c % 2], gbsems[c % 2])

        def scat(c):
            return pltpu.async_copy(
                abufs[c % 3], out_hbm.at[pl.ds(base + c * CH, CH)],
                sasems[c % 3])

        ga = {0: gath_a(0)}
        gb = {0: gath_b(0)}
        so = {}
        for c in range(NCK):
            if c + 1 < NCK:
                if c - 2 in so:
                    so[c - 2].wait()
                ga[c + 1] = gath_a(c + 1)
                gb[c + 1] = gath_b(c + 1)
            ga[c].wait()
            gb[c].wait()
            av = abufs[c % 3]
            bv = bbufs[c % 2]
            wq0 = w0_v[c // 2]
            wq1 = w1_v[c // 2]
            dn = lax.GatherDimensionNumbers(
                offset_dims=(), collapsed_slice_dims=(0,),
                start_index_map=(0,))
            spl0 = []
            spl1 = []
            for r in range(CH):
                ivec = jnp.full((16, 1), (c % 2) * CH + r, jnp.int32)
                spl0.append(lax.gather(
                    wq0, ivec, dn, (1,),
                    mode=lax.GatherScatterMode.PROMISE_IN_BOUNDS))
                spl1.append(lax.gather(
                    wq1, ivec, dn, (1,),
                    mode=lax.GatherScatterMode.PROMISE_IN_BOUNDS))

            def vadd(k, _):
                for r in range(CH):
                    av[r, pl.ds(k * 16, 16)] = (
                        av[r, pl.ds(k * 16, 16)] * spl0[r]
                        + bv[r, pl.ds(k * 16, 16)] * spl1[r])
                return 0
            lax.fori_loop(0, D // 16, vadd, 0)
            so[c] = scat(c)
        so[NCK - 3].wait()
        so[NCK - 2].wait()
        so[NCK - 1].wait()

    return body


def _sc_combine(y_sorted, s0, s1, w0, w1):
    return _sc_combine_kernel()(y_sorted, s0, s1, w0, w1)


def _gmm_body(meta_ref, xs_ref, wg_ref, wu_ref, wd_ref, y_ref):
    b = pl.program_id(0)

    @pl.when(meta_ref[b, 1] == 1)
    def _():
        xb = xs_ref[...]
        wg = wg_ref[0]
        wu = wu_ref[0]
        wd = wd_ref[0]
        g = lax.dot_general(xb, wg, (((1,), (1,)), ((), ())),
                            preferred_element_type=jnp.float32)
        u = lax.dot_general(xb, wu, (((1,), (1,)), ((), ())),
                            preferred_element_type=jnp.float32)
        h = g * jax.nn.sigmoid(g) * u
        y_ref[...] = lax.dot_general(h, wd, (((1,), (1,)), ((), ())),
                                     preferred_element_type=jnp.float32)


def _gmm(meta, x_sorted, wg, wu, wd):
    grid_spec = pltpu.PrefetchScalarGridSpec(
        num_scalar_prefetch=1,
        grid=(NB,),
        in_specs=[
            pl.BlockSpec((BR, D), lambda b, meta: (b, 0)),
            pl.BlockSpec((1, FF, D), lambda b, meta: (meta[b, 0], 0, 0)),
            pl.BlockSpec((1, FF, D), lambda b, meta: (meta[b, 0], 0, 0)),
            pl.BlockSpec((1, D, FF), lambda b, meta: (meta[b, 0], 0, 0)),
        ],
        out_specs=pl.BlockSpec((BR, D), lambda b, meta: (b, 0)),
    )
    return pl.pallas_call(
        _gmm_body,
        grid_spec=grid_spec,
        out_shape=jax.ShapeDtypeStruct((NPAD, D), jnp.float32),
    )(meta, x_sorted, wg, wu, wd)


def kernel(hidden_states, gate_w, w_gate_proj, w_up_proj, w_down_proj):
    x = hidden_states.reshape(T, D)
    logits, w0r, w1r, s0r, s1r, meta = _router(x, gate_w)

    x_sorted = _sc_dispatch(x, s0r, s1r)
    y_sorted = _gmm(meta, x_sorted, w_gate_proj, w_up_proj, w_down_proj)
    final = _sc_combine(y_sorted, s0r, s1r, w0r, w1r)
    return (final.reshape(B, S, D), logits)


# speedup vs baseline: 1.0085x; 1.0085x over previous
import functools

import jax
import jax.numpy as jnp
from jax import lax
from jax.experimental import pallas as pl
from jax.experimental.pallas import tpu as pltpu
from jax.experimental.pallas import tpu_sc as plsc

B, S, D = 1, 2048, 2048
E, TOPK, FF = 8, 2, 1024
T = B * S
BR = 256
NB = (T * TOPK) // BR + (E - 1)
NPAD = NB * BR
NBP = 128
CK = 256

NC, NS = 2, 16
NW = NC * NS
TPW = T // NW
CH = 8
NCK = TPW // CH


def _router_body(x_ref, gw_ref, logits_ref, w0_ref, w1_ref,
                 s0_ref, s1_ref, meta_ref):
    x = x_ref[...]
    gw = gw_ref[...]
    logits = lax.dot_general(x, gw, (((1,), (1,)), ((), ())),
                             preferred_element_type=jnp.float32)
    logits_ref[...] = logits
    m = jnp.max(logits, axis=1, keepdims=True)
    ex = jnp.exp(logits - m)
    p = ex / jnp.sum(ex, axis=1, keepdims=True)
    a1 = jnp.argmax(p, axis=1).astype(jnp.int32)
    w1 = jnp.max(p, axis=1)
    cols = lax.broadcasted_iota(jnp.int32, p.shape, 1)
    p2 = jnp.where(cols == a1[:, None], -jnp.inf, p)
    a2 = jnp.argmax(p2, axis=1).astype(jnp.int32)
    w2 = jnp.max(p2, axis=1)
    w0_ref[...] = w1.reshape(T // 16, 16)
    w1_ref[...] = w2.reshape(T // 16, 16)

    oh1 = (cols == a1[:, None]).astype(jnp.float32)
    oh2 = (cols == a2[:, None]).astype(jnp.float32)
    cmb = oh1 + oh2

    r_i = lax.broadcasted_iota(jnp.int32, (CK, CK), 0)
    c_i = lax.broadcasted_iota(jnp.int32, (CK, CK), 1)
    tri = (r_i >= c_i).astype(jnp.float32)
    parts = []
    run = jnp.zeros((1, E), jnp.float32)
    for i in range(T // CK):
        blk = cmb[i * CK:(i + 1) * CK, :]
        s = lax.dot_general(tri, blk, (((1,), (0,)), ((), ())),
                            preferred_element_type=jnp.float32) + run
        parts.append(s)
        run = s[CK - 1:CK, :]
    csum = jnp.concatenate(parts, axis=0)
    excl = csum - cmb

    counts = run
    padded = jnp.ceil(counts / BR) * BR
    e_r = lax.broadcasted_iota(jnp.int32, (E, E), 0)
    e_c = lax.broadcasted_iota(jnp.int32, (E, E), 1)
    m8 = (e_r <= e_c).astype(jnp.float32)
    csum_p = lax.dot_general(padded, m8, (((1,), (0,)), ((), ())),
                             preferred_element_type=jnp.float32)
    offs_p = csum_p - padded

    base = excl + offs_p
    s0 = jnp.sum(oh1 * base, axis=1)
    s1 = jnp.sum(oh2 * base, axis=1)
    s0_ref[...] = s0.reshape(T // CH, CH).astype(jnp.int32)
    s1_ref[...] = s1.reshape(T // CH, CH).astype(jnp.int32)

    total = csum_p[0:1, E - 1:E]
    bstart = (lax.broadcasted_iota(jnp.int32, (NBP, 1), 0)
              .astype(jnp.float32) * BR)
    be = jnp.sum((bstart >= csum_p).astype(jnp.float32), axis=1,
                 keepdims=True)
    act = (bstart < total).astype(jnp.float32)
    be_last = jnp.sum((csum_p <= total - 1.0).astype(jnp.float32),
                      axis=1, keepdims=True)
    be_last = jnp.minimum(be_last, float(E - 1))
    be = jnp.where(act > 0, be, be_last)
    meta_ref[...] = jnp.concatenate([be, act], axis=1).astype(jnp.int32)


def _router(x, gate_w):
    return pl.pallas_call(
        _router_body,
        out_shape=(
            jax.ShapeDtypeStruct((T, E), jnp.float32),
            jax.ShapeDtypeStruct((T // 16, 16), jnp.float32),
            jax.ShapeDtypeStruct((T // 16, 16), jnp.float32),
            jax.ShapeDtypeStruct((T // CH, CH), jnp.int32),
            jax.ShapeDtypeStruct((T // CH, CH), jnp.int32),
            jax.ShapeDtypeStruct((NBP, 2), jnp.int32),
        ),
    )(x, gate_w)


@functools.lru_cache(maxsize=None)
def _sc_dispatch_kernel():
    mesh = plsc.VectorSubcoreMesh(core_axis_name="c", subcore_axis_name="s")

    @functools.partial(
        pl.kernel, mesh=mesh,
        out_type=jax.ShapeDtypeStruct((NPAD, D), jnp.float32),
        scratch_types=[
            pltpu.VMEM((NCK, CH), jnp.int32),
            pltpu.VMEM((NCK, CH), jnp.int32),
            pltpu.VMEM((CH, D), jnp.float32),
            pltpu.VMEM((CH, D), jnp.float32),
            pltpu.VMEM((CH, D), jnp.float32),
            pltpu.SemaphoreType.DMA,
            pltpu.SemaphoreType.DMA,
            pltpu.SemaphoreType.DMA,
            pltpu.SemaphoreType.DMA,
            pltpu.SemaphoreType.DMA,
            pltpu.SemaphoreType.DMA,
            pltpu.SemaphoreType.DMA,
            pltpu.SemaphoreType.DMA,
            pltpu.SemaphoreType.DMA,
            pltpu.SemaphoreType.DMA,
        ],
    )
    def body(x_hbm, s0_hbm, s1_hbm, out_hbm, i0_v, i1_v,
             r0, r1, r2, g0, g1, g2, sa0, sa1, sa2, sb0, sb1, sb2, si):
        wid = lax.axis_index("s") * NC + lax.axis_index("c")
        base = wid * TPW
        rbase = wid * NCK
        ic0 = pltpu.async_copy(s0_hbm.at[pl.ds(rbase, NCK), :], i0_v, si)
        ic1 = pltpu.async_copy(s1_hbm.at[pl.ds(rbase, NCK), :], i1_v, si)
        bufs = (r0, r1, r2)
        gsems = (g0, g1, g2)
        sasems = (sa0, sa1, sa2)
        sbsems = (sb0, sb1, sb2)

        def rd(c):
            return pltpu.async_copy(
                x_hbm.at[pl.ds(base + c * CH, CH), :],
                bufs[c % 3], gsems[c % 3])

        def wr(c):
            return (pltpu.async_copy(bufs[c % 3], out_hbm.at[i0_v.at[c]],
                                     sasems[c % 3]),
                    pltpu.async_copy(bufs[c % 3], out_hbm.at[i1_v.at[c]],
                                     sbsems[c % 3]))

        gd = {0: rd(0)}
        sd = {}
        ic0.wait()
        ic1.wait()
        for c in range(NCK):
            if c + 1 < NCK:
                if c - 2 in sd:
                    sd[c - 2][0].wait()
                    sd[c - 2][1].wait()
                gd[c + 1] = rd(c + 1)
            gd[c].wait()
            sd[c] = wr(c)
        for c in (NCK - 3, NCK - 2, NCK - 1):
            sd[c][0].wait()
            sd[c][1].wait()

    return body


def _sc_dispatch(x, s0r, s1r):
    return _sc_dispatch_kernel()(x, s0r, s1r)


@functools.lru_cache(maxsize=None)
def _sc_combine_kernel():
    mesh = plsc.VectorSubcoreMesh(core_axis_name="c", subcore_axis_name="s")

    @functools.partial(
        pl.kernel, mesh=mesh,
        out_type=jax.ShapeDtypeStruct((T, D), jnp.float32),
        scratch_types=[
            pltpu.VMEM((NCK, CH), jnp.int32),
            pltpu.VMEM((NCK, CH), jnp.int32),
            pltpu.VMEM((TPW // 16, 16), jnp.float32),
            pltpu.VMEM((TPW // 16, 16), jnp.float32),
            pltpu.VMEM((CH, D), jnp.float32),
            pltpu.VMEM((CH, D), jnp.float32),
            pltpu.VMEM((CH, D), jnp.float32),
            pltpu.VMEM((CH, D), jnp.float32),
            pltpu.VMEM((CH, D), jnp.float32),
            pltpu.SemaphoreType.DMA,
            pltpu.SemaphoreType.DMA,
            pltpu.SemaphoreType.DMA,
            pltpu.SemaphoreType.DMA,
            pltpu.SemaphoreType.DMA,
            pltpu.SemaphoreType.DMA,
            pltpu.SemaphoreType.DMA,
            pltpu.SemaphoreType.DMA,
            pltpu.SemaphoreType.DMA,
        ],
    )
    def body(y_hbm, s0_hbm, s1_hbm, w0_hbm, w1_hbm, out_hbm, i0_v, i1_v,
             w0_v, w1_v, a0, a1, a2, b0, b1,
             ga0, ga1, ga2, gb0, gb1, sa0, sa1, sa2, si):
        wid = lax.axis_index("s") * NC + lax.axis_index("c")
        base = wid * TPW
        rbase = wid * NCK
        qbase = wid * (TPW // 16)
        ics = [
            pltpu.async_copy(s0_hbm.at[pl.ds(rbase, NCK), :], i0_v, si),
            pltpu.async_copy(s1_hbm.at[pl.ds(rbase, NCK), :], i1_v, si),
            pltpu.async_copy(w0_hbm.at[pl.ds(qbase, TPW // 16), :], w0_v, si),
            pltpu.async_copy(w1_hbm.at[pl.ds(qbase, TPW // 16), :], w1_v, si),
        ]
        for ic in ics:
            ic.wait()
        abufs = (a0, a1, a2)
        bbufs = (b0, b1)
        gasems = (ga0, ga1, ga2)
        gbsems = (gb0, gb1)
        sasems = (sa0, sa1, sa2)

        def gath_a(c):
            return pltpu.async_copy(
                y_hbm.at[i0_v.at[c]],
                abufs[c % 3], gasems[c % 3])

        def gath_b(c):
            return pltpu.async_copy(
                y_hbm.at[i1_v.at[c]],
                bbufs[c % 2], gbsems[c % 2])

        def scat(c):
            return pltpu.async_copy(
                abufs[c % 3], out_hbm.at[pl.ds(base + c * CH, CH)],
                sasems[c % 3])

        ga = {0: gath_a(0)}
        gb = {0: gath_b(0)}
        so = {}
        for c in range(NCK):
            if c + 1 < NCK:
                if c - 2 in so:
                    so[c - 2].wait()
                ga[c + 1] = gath_a(c + 1)
                gb[c + 1] = gath_b(c + 1)
            ga[c].wait()
            gb[c].wait()
            av = abufs[c % 3]
            bv = bbufs[c % 2]
            wq0 = w0_v[c // 2]
            wq1 = w1_v[c // 2]
            dn = lax.GatherDimensionNumbers(
                offset_dims=(), collapsed_slice_dims=(0,),
                start_index_map=(0,))
            spl0 = []
            spl1 = []
            for r in range(CH):
                ivec = jnp.full((16, 1), (c % 2) * CH + r, jnp.int32)
                spl0.append(lax.gather(
                    wq0, ivec, dn, (1,),
                    mode=lax.GatherScatterMode.PROMISE_IN_BOUNDS))
                spl1.append(lax.gather(
                    wq1, ivec, dn, (1,),
                    mode=lax.GatherScatterMode.PROMISE_IN_BOUNDS))

            def vadd(k, _):
                for r in range(CH):
                    av[r, pl.ds(k * 16, 16)] = (
                        av[r, pl.ds(k * 16, 16)] * spl0[r]
                        + bv[r, pl.ds(k * 16, 16)] * spl1[r])
                return 0
            lax.fori_loop(0, D // 16, vadd, 0)
            so[c] = scat(c)
        so[NCK - 3].wait()
        so[NCK - 2].wait()
        so[NCK - 1].wait()

    return body


def _sc_combine(y_sorted, s0, s1, w0, w1):
    return _sc_combine_kernel()(y_sorted, s0, s1, w0, w1)


def _gmm_body(meta_ref, xs_ref, wg_ref, wu_ref, wd_ref, y_ref):
    b = pl.program_id(0)

    @pl.when(meta_ref[b, 1] == 1)
    def _():
        xb = xs_ref[...]
        wg = wg_ref[0]
        wu = wu_ref[0]
        wd = wd_ref[0]
        g = lax.dot_general(xb, wg, (((1,), (1,)), ((), ())),
                            preferred_element_type=jnp.float32)
        u = lax.dot_general(xb, wu, (((1,), (1,)), ((), ())),
                            preferred_element_type=jnp.float32)
        h = g * jax.nn.sigmoid(g) * u
        y_ref[...] = lax.dot_general(h, wd, (((1,), (1,)), ((), ())),
                                     preferred_element_type=jnp.float32)


def _gmm(meta, x_sorted, wg, wu, wd):
    grid_spec = pltpu.PrefetchScalarGridSpec(
        num_scalar_prefetch=1,
        grid=(NB,),
        in_specs=[
            pl.BlockSpec((BR, D), lambda b, meta: (b, 0)),
            pl.BlockSpec((1, FF, D), lambda b, meta: (meta[b, 0], 0, 0)),
            pl.BlockSpec((1, FF, D), lambda b, meta: (meta[b, 0], 0, 0)),
            pl.BlockSpec((1, D, FF), lambda b, meta: (meta[b, 0], 0, 0)),
        ],
        out_specs=pl.BlockSpec((BR, D), lambda b, meta: (b, 0)),
    )
    return pl.pallas_call(
        _gmm_body,
        grid_spec=grid_spec,
        out_shape=jax.ShapeDtypeStruct((NPAD, D), jnp.float32),
    )(meta, x_sorted, wg, wu, wd)


def kernel(hidden_states, gate_w, w_gate_proj, w_up_proj, w_down_proj):
    x = hidden_states.reshape(T, D)
    logits, w0r, w1r, s0r, s1r, meta = _router(x, gate_w)

    x_sorted = _sc_dispatch(x, s0r, s1r)
    y_sorted = _gmm(meta, x_sorted, w_gate_proj, w_up_proj, w_down_proj)
    final = _sc_combine(y_sorted, s0r, s1r, w0r, w1r)
    return (final.reshape(B, S, D), logits)
